# Initial kernel scaffold; baseline (speedup 1.0000x reference)
import jax
import jax.numpy as jnp
from jax.experimental import pallas as pl


def kernel(concrete_lower, concrete_upper, abstract_lower, abstract_upper, conv_weight, conv_bias, M, box_lower, box_upper):
    raise NotImplementedError("write your pallas kernel here")


# speedup vs baseline: 1.6179x; 1.6179x over previous
import functools

import jax
import jax.numpy as jnp
from jax.experimental import pallas as pl

_C, _H, _W = 3, 32, 32
_OC, _OH, _OW = 16, 16, 16
_KH, _KW = 4, 4
_PREV = _C * _H * _W
_OUT = _OC * _OH * _OW
_ROWS_A = _PREV + 1
_COLS_A = _OUT + 1
_D_IN = 1025

_A_BLK_R, _A_BLK_C = 1024, 256
_N_TILE = 512


def _a_build_kernel(w_ref, out_ref):
    i = pl.program_id(0)
    r = jax.lax.broadcasted_iota(jnp.int32, (_A_BLK_R, _A_BLK_C), 0)
    s = jax.lax.broadcasted_iota(jnp.int32, (_A_BLK_R, _A_BLK_C), 1)
    ih = r // _W
    iw = r % _W
    oh = s // _OW
    ow = s % _OW
    khv = ih - 2 * oh + 1
    kwv = iw - 2 * ow + 1
    val = jnp.zeros((_A_BLK_R, _A_BLK_C), jnp.float32)
    for kh in range(_KH):
        t = jnp.zeros((_A_BLK_R, _A_BLK_C), jnp.float32)
        for kw in range(_KW):
            t = jnp.where(kwv == kw, w_ref[0, 0, kh, kw], t)
        val = jnp.where(khv == kh, t, val)
    j = pl.program_id(1)
    p = i * _A_BLK_R + r
    o = j * _A_BLK_C + s
    val = jnp.where((p < _PREV) & (o < _OUT), val, 0.0)
    out_ref[...] = val


def _build_a(conv_weight, conv_bias):
    grid = (pl.cdiv(_ROWS_A, _A_BLK_R), pl.cdiv(_COLS_A, _A_BLK_C))
    a = pl.pallas_call(
        _a_build_kernel,
        grid=grid,
        in_specs=[
            pl.BlockSpec(
                (1, 1, _KH, _KW),
                lambda i, j: (jnp.minimum(j, _OC - 1), jnp.minimum(i, _C - 1), 0, 0),
            ),
        ],
        out_specs=pl.BlockSpec((_A_BLK_R, _A_BLK_C), lambda i, j: (i, j)),
        out_shape=jax.ShapeDtypeStruct((_ROWS_A, _COLS_A), jnp.float32),
    )(conv_weight)
    bias_row = jnp.concatenate(
        [jnp.repeat(conv_bias, _OUT // _OC), jnp.ones((1,), jnp.float32)]
    )
    return a.at[_PREV, :].set(bias_row)


def _bounds_kernel(m_ref, a_ref, ce_ref, re_ref, low_ref, up_ref):
    b = jnp.dot(m_ref[...], a_ref[...], preferred_element_type=jnp.float32)
    t1 = jnp.dot(ce_ref[...], b, preferred_element_type=jnp.float32)
    t2 = jnp.dot(re_ref[...], jnp.abs(b), preferred_element_type=jnp.float32)
    low_ref[...] = t1 - t2
    up_ref[...] = t1 + t2


def _bounds(m, a, ce, re):
    n_tiles = pl.cdiv(_COLS_A, _N_TILE)
    low, up = pl.pallas_call(
        _bounds_kernel,
        grid=(n_tiles,),
        in_specs=[
            pl.BlockSpec((_D_IN, _ROWS_A), lambda n: (0, 0)),
            pl.BlockSpec((_ROWS_A, _N_TILE), lambda n: (0, n)),
            pl.BlockSpec((1, _D_IN), lambda n: (0, 0)),
            pl.BlockSpec((1, _D_IN), lambda n: (0, 0)),
        ],
        out_specs=[
            pl.BlockSpec((1, _N_TILE), lambda n: (0, n)),
            pl.BlockSpec((1, _N_TILE), lambda n: (0, n)),
        ],
        out_shape=[
            jax.ShapeDtypeStruct((1, n_tiles * _N_TILE), jnp.float32),
            jax.ShapeDtypeStruct((1, n_tiles * _N_TILE), jnp.float32),
        ],
    )(m, a, ce, re)
    return low, up


@jax.jit
def kernel(concrete_lower, concrete_upper, abstract_lower, abstract_upper,
           conv_weight, conv_bias, M, box_lower, box_upper):
    a = _build_a(conv_weight, conv_bias)
    c = (box_lower + box_upper) * 0.5
    r = (box_upper - box_lower) * 0.5
    ce = jnp.concatenate([c, jnp.ones((1,), jnp.float32)])[None, :]
    re = jnp.concatenate([r, jnp.zeros((1,), jnp.float32)])[None, :]
    low, up = _bounds(M, a, ce, re)
    out_dim = (_OC, _OH, _OW)
    lower_out = low[0, :_OUT].reshape(out_dim)
    upper_out = up[0, :_OUT].reshape(out_dim)
    return (lower_out, upper_out, a, a)
